# baseline (speedup 1.0000x reference)
import functools

import jax
import jax.numpy as jnp
from jax import lax
from jax.experimental import pallas as pl
from jax.experimental.pallas import tpu as pltpu
from jax.experimental.pallas import tpu_sc as plsc

NUM_EMBEDDINGS = 1000000
D = 64
B_TOTAL = 4096 * 200

NC, NS = 2, 16
NW = NC * NS
L = 16

N_PER_W = B_TOTAL // NW
C = 128
NCH = N_PER_W // C
S = 4
A = 2
P = 2 * S

assert NCH % P == 0 and S == 2 * A


@functools.partial(
    pl.kernel,
    out_type=jax.ShapeDtypeStruct((B_TOTAL, D), jnp.float32),
    mesh=plsc.VectorSubcoreMesh(core_axis_name="c", subcore_axis_name="s"),
    compiler_params=pltpu.CompilerParams(
        use_tc_tiling_on_sc=True, needs_layout_passes=False
    ),
    scratch_types=[
        [pltpu.VMEM((C,), jnp.int32) for _ in range(P)],
        [pltpu.VMEM((C,), jnp.int32) for _ in range(S)],
        pltpu.VMEM((C,), jnp.int32),
        [pltpu.SemaphoreType.DMA for _ in range(P)],
        [pltpu.VMEM((C, 2 * D), jnp.float32) for _ in range(S)],
        [pltpu.VMEM((C, D), jnp.float32) for _ in range(2)],
        [pltpu.SemaphoreType.DMA for _ in range(S)],
        [pltpu.SemaphoreType.DMA for _ in range(2)],
    ],
)
def _gather_sc(idx_hbm, table_hbm, out_hbm, ixbufs, rowbufs, cb_v, isems,
               gbufs, obufs, gsems, wsems):
    wid = lax.axis_index("s") * NC + lax.axis_index("c")
    base = pl.multiple_of(wid * N_PER_W, N_PER_W)

    def _idx_start(g, xs):
        r = pl.multiple_of(base + g * C, C)
        pltpu.make_async_copy(idx_hbm.at[pl.ds(r, C)], ixbufs[xs], isems[xs]).start()

    def _idx_wait(g, xs):
        r = pl.multiple_of(base + g * C, C)
        pltpu.make_async_copy(idx_hbm.at[pl.ds(r, C)], ixbufs[xs], isems[xs]).wait()

    def _gather_start(g, s, xs):
        _idx_wait(g, xs)
        ix = ixbufs[xs]

        def rbody(k, _):
            v = ix[pl.ds(k * L, L)]
            rowbufs[s][pl.ds(k * L, L)] = lax.shift_right_logical(v, 1)
            return _

        lax.fori_loop(0, C // L, rbody, 0, unroll=4)
        pltpu.make_async_copy(table_hbm.at[rowbufs[s]], gbufs[s], gsems[s]).start()

    def _gather_wait(g, s):
        pltpu.make_async_copy(table_hbm.at[rowbufs[s]], gbufs[s], gsems[s]).wait()

    def _write_start(g, so):
        r = pl.multiple_of(base + g * C, C)
        pltpu.make_async_copy(obufs[so], out_hbm.at[pl.ds(r, C)], wsems[so]).start()

    def _write_wait(g, so):
        r = pl.multiple_of(base + g * C, C)
        pltpu.make_async_copy(obufs[so], out_hbm.at[pl.ds(r, C)], wsems[so]).wait()

    iota = lax.iota(jnp.int32, L)

    def _extract(g, s, so, xs):
        ix = ixbufs[xs]

        def cbody(k, _):
            v = ix[pl.ds(k * L, L)]
            cb_v[pl.ds(k * L, L)] = lax.shift_left(lax.bitwise_and(v, 1), 6)
            return _

        lax.fori_loop(0, C // L, cbody, 0, unroll=4)

        def ebody(j, _):
            cb = plsc.load_gather(cb_v, [jnp.broadcast_to(j, (L,))])
            for v in range(D // L):
                vals = plsc.load_gather(
                    gbufs[s], [jnp.broadcast_to(j, (L,)), cb + (iota + v * L)]
                )
                obufs[so][j, pl.ds(v * L, L)] = vals
            return _

        lax.fori_loop(0, C, ebody, 0, unroll=8)

    for g in range(2 * A):
        _idx_start(g, g % P)
    for g in range(A):
        _gather_start(g, g % S, g % P)
    for g in range(P):
        if g + 2 * A < NCH:
            _idx_start(g + 2 * A, (g + 2 * A) % P)
        if g + A < NCH:
            _gather_start(g + A, (g + A) % S, (g + A) % P)
        _gather_wait(g, g % S)
        if g >= 2:
            _write_wait(g - 2, g % 2)
        _extract(g, g % S, g % 2, g % P)
        _write_start(g, g % 2)

    def trip(i, _):
        g0 = i * P
        for b in range(P):
            g = g0 + b
            _idx_start(g + 2 * A, (b + 2 * A) % P)
            _gather_start(g + A, (b + A) % S, (b + A) % P)
            _gather_wait(g, b % S)
            _write_wait(g - 2, b % 2)
            _extract(g, b % S, b % 2, b)
            _write_start(g, b % 2)
        return _

    lax.fori_loop(1, NCH // P - 1, trip, 0)

    g0 = NCH - P
    for b in range(P):
        g = g0 + b
        if g + 2 * A < NCH:
            _idx_start(g + 2 * A, (b + 2 * A) % P)
        if g + A < NCH:
            _gather_start(g + A, (b + A) % S, (b + A) % P)
        _gather_wait(g, b % S)
        _write_wait(g - 2, b % 2)
        _extract(g, b % S, b % 2, b)
        _write_start(g, b % 2)
    for b in range(2):
        _write_wait(NCH - 2 + b, (NCH - 2 + b) % 2)


def kernel(token_ids, embedding_matrix):
    idx = token_ids.reshape(-1)
    table2 = embedding_matrix.reshape(NUM_EMBEDDINGS // 2, 2 * D)
    out = _gather_sc(idx, table2)
    return out.reshape(token_ids.shape[0], token_ids.shape[1], D)


# speedup vs baseline: 1.2514x; 1.2514x over previous
import functools

import jax
import jax.numpy as jnp
from jax import lax
from jax.experimental import pallas as pl
from jax.experimental.pallas import tpu as pltpu
from jax.experimental.pallas import tpu_sc as plsc

NUM_EMBEDDINGS = 1000000
D = 64
B_TOTAL = 4096 * 200

NC, NS = 2, 16
NW = NC * NS
L = 16

N_PER_W = B_TOTAL // NW
C = 128
NCH = N_PER_W // C
S = 4
A = 2
P = 2 * S

assert NCH % P == 0 and S == 2 * A


@functools.partial(
    pl.kernel,
    out_type=jax.ShapeDtypeStruct((B_TOTAL, D), jnp.float32),
    mesh=plsc.VectorSubcoreMesh(core_axis_name="c", subcore_axis_name="s"),
    compiler_params=pltpu.CompilerParams(
        use_tc_tiling_on_sc=True, needs_layout_passes=False
    ),
    scratch_types=[
        [pltpu.VMEM((C,), jnp.int32) for _ in range(P)],
        [pltpu.VMEM((C,), jnp.int32) for _ in range(S)],
        pltpu.VMEM((C,), jnp.int32),
        [pltpu.SemaphoreType.DMA for _ in range(P)],
        [pltpu.VMEM((C, 2 * D), jnp.float32) for _ in range(S)],
        [pltpu.VMEM((C, D), jnp.float32) for _ in range(2)],
        [pltpu.SemaphoreType.DMA for _ in range(S)],
        [pltpu.SemaphoreType.DMA for _ in range(2)],
    ],
)
def _gather_sc(idx_hbm, table_hbm, out_hbm, ixbufs, rowbufs, cb_v, isems,
               gbufs, obufs, gsems, wsems):
    wid = lax.axis_index("s") * NC + lax.axis_index("c")
    base = pl.multiple_of(wid * N_PER_W, N_PER_W)

    def _idx_start(g, xs):
        r = pl.multiple_of(base + g * C, C)
        pltpu.make_async_copy(idx_hbm.at[pl.ds(r, C)], ixbufs[xs], isems[xs]).start()

    def _idx_wait(g, xs):
        r = pl.multiple_of(base + g * C, C)
        pltpu.make_async_copy(idx_hbm.at[pl.ds(r, C)], ixbufs[xs], isems[xs]).wait()

    def _gather_start(g, s, xs):
        _idx_wait(g, xs)
        ix = ixbufs[xs]

        def rbody(k, _):
            v = ix[pl.ds(k * L, L)]
            rowbufs[s][pl.ds(k * L, L)] = lax.shift_right_logical(v, 1)
            return _

        lax.fori_loop(0, C // L, rbody, 0, unroll=4)
        pltpu.make_async_copy(table_hbm.at[rowbufs[s]], gbufs[s], gsems[s]).start()

    def _gather_wait(g, s):
        pltpu.make_async_copy(table_hbm.at[rowbufs[s]], gbufs[s], gsems[s]).wait()

    def _write_start(g, so):
        r = pl.multiple_of(base + g * C, C)
        pltpu.make_async_copy(obufs[so], out_hbm.at[pl.ds(r, C)], wsems[so]).start()

    def _write_wait(g, so):
        r = pl.multiple_of(base + g * C, C)
        pltpu.make_async_copy(obufs[so], out_hbm.at[pl.ds(r, C)], wsems[so]).wait()

    iota = lax.iota(jnp.int32, L)

    def _extract(g, s, so, xs):
        ix = ixbufs[xs]

        def cbody(k, _):
            v = ix[pl.ds(k * L, L)]
            cb_v[pl.ds(k * L, L)] = lax.shift_left(lax.bitwise_and(v, 1), 6)
            return _

        lax.fori_loop(0, C // L, cbody, 0, unroll=4)

        @plsc.parallel_loop(0, C, unroll=8)
        def ebody(j):
            cb = plsc.load_gather(cb_v, [jnp.broadcast_to(j, (L,))])
            for v in range(D // L):
                vals = plsc.load_gather(
                    gbufs[s], [jnp.broadcast_to(j, (L,)), cb + (iota + v * L)]
                )
                obufs[so][j, pl.ds(v * L, L)] = vals

    for g in range(2 * A):
        _idx_start(g, g % P)
    for g in range(A):
        _gather_start(g, g % S, g % P)
    for g in range(P):
        if g + 2 * A < NCH:
            _idx_start(g + 2 * A, (g + 2 * A) % P)
        if g + A < NCH:
            _gather_start(g + A, (g + A) % S, (g + A) % P)
        _gather_wait(g, g % S)
        if g >= 2:
            _write_wait(g - 2, g % 2)
        _extract(g, g % S, g % 2, g % P)
        _write_start(g, g % 2)

    def trip(i, _):
        g0 = i * P
        for b in range(P):
            g = g0 + b
            _idx_start(g + 2 * A, (b + 2 * A) % P)
            _gather_start(g + A, (b + A) % S, (b + A) % P)
            _gather_wait(g, b % S)
            _write_wait(g - 2, b % 2)
            _extract(g, b % S, b % 2, b)
            _write_start(g, b % 2)
        return _

    lax.fori_loop(1, NCH // P - 1, trip, 0)

    g0 = NCH - P
    for b in range(P):
        g = g0 + b
        if g + 2 * A < NCH:
            _idx_start(g + 2 * A, (b + 2 * A) % P)
        if g + A < NCH:
            _gather_start(g + A, (b + A) % S, (b + A) % P)
        _gather_wait(g, b % S)
        _write_wait(g - 2, b % 2)
        _extract(g, b % S, b % 2, b)
        _write_start(g, b % 2)
    for b in range(2):
        _write_wait(NCH - 2 + b, (NCH - 2 + b) % 2)


def kernel(token_ids, embedding_matrix):
    idx = token_ids.reshape(-1)
    table2 = embedding_matrix.reshape(NUM_EMBEDDINGS // 2, 2 * D)
    out = _gather_sc(idx, table2)
    return out.reshape(token_ids.shape[0], token_ids.shape[1], D)
